# Initial kernel scaffold; baseline (speedup 1.0000x reference)
import jax
import jax.numpy as jnp
from jax.experimental import pallas as pl


def kernel(position_ids, base, alpha, beta):
    raise NotImplementedError("write your pallas kernel here")


# speedup vs baseline: 3.8270x; 3.8270x over previous
import functools
import math

import jax
import jax.numpy as jnp
import numpy as np
from jax import lax
from jax.experimental import pallas as pl
from jax.experimental.pallas import tpu as pltpu
from jax.experimental.pallas import tpu_sc as plsc

NUM_EMBEDDINGS = 8192
EMBEDDING_DIM = 1024
HALF_DIM = EMBEDDING_DIM // 2
_DT = math.log(10000.0) / (HALF_DIM - 1)
_DIV_FULL = np.exp(-_DT * (np.arange(EMBEDDING_DIM) // 2)).astype(np.float32)[None, :]

_ROWS_BLK = 1024
_GRID = NUM_EMBEDDINGS // _ROWS_BLK


def _table_body(base_ref, beta_ref, div_ref, out_ref):
    i = pl.program_id(0)
    rows = jax.lax.broadcasted_iota(jnp.int32, (_ROWS_BLK, EMBEDDING_DIM), 0)
    rows = (rows + i * _ROWS_BLK).astype(jnp.float32)
    freq = base_ref[0, 0] * div_ref[...] + beta_ref[0, 0]
    ang = rows * freq
    col = jax.lax.broadcasted_iota(jnp.int32, (_ROWS_BLK, EMBEDDING_DIM), 1)
    out_ref[...] = jnp.where(col % 2 == 1, jnp.cos(ang), 0.0)


def _build_table(base, beta):
    return pl.pallas_call(
        _table_body,
        grid=(_GRID,),
        in_specs=[
            pl.BlockSpec(memory_space=pltpu.SMEM),
            pl.BlockSpec(memory_space=pltpu.SMEM),
            pl.BlockSpec((1, EMBEDDING_DIM), lambda i: (0, 0)),
        ],
        out_specs=pl.BlockSpec((_ROWS_BLK, EMBEDDING_DIM), lambda i: (i, 0)),
        out_shape=jax.ShapeDtypeStruct((NUM_EMBEDDINGS, EMBEDDING_DIM), jnp.float32),
    )(base.reshape(1, 1), beta.reshape(1, 1), jnp.asarray(_DIV_FULL))


_NW = 32
_B = 4 * 8192
_BPW = _B // _NW
_C = 32
_NCHUNK = _BPW // _C


@functools.cache
def _make_gather():
    mesh = plsc.VectorSubcoreMesh(core_axis_name="c", subcore_axis_name="s")

    @functools.partial(
        pl.kernel,
        mesh=mesh,
        out_type=jax.ShapeDtypeStruct((_B, EMBEDDING_DIM), jnp.float32),
        scratch_types=[
            pltpu.VMEM((_NCHUNK, _C), jnp.int32),
            pltpu.VMEM((_C, EMBEDDING_DIM), jnp.float32),
            pltpu.VMEM((_C, EMBEDDING_DIM), jnp.float32),
            pltpu.SemaphoreType.DMA,
        ],
    )
    def gather(table_hbm, idx_hbm, out_hbm, idx_v, buf0, buf1, gsem):
        wid = lax.axis_index("s") * 2 + lax.axis_index("c")
        base_row = wid * _BPW
        pltpu.sync_copy(idx_hbm.at[wid], idx_v)
        bufs = (buf0, buf1)
        pltpu.async_copy(table_hbm.at[idx_v.at[0]], buf0, gsem).wait()
        for j in range(_NCHUNK):
            cur = bufs[j % 2]
            nxt = bufs[(j + 1) % 2]
            if j + 1 < _NCHUNK:
                g = pltpu.async_copy(table_hbm.at[idx_v.at[j + 1]], nxt, gsem)
            pltpu.sync_copy(cur, out_hbm.at[pl.ds(base_row + j * _C, _C)])
            if j + 1 < _NCHUNK:
                g.wait()

    return gather


def kernel(position_ids, base, alpha, beta):
    del alpha
    batch, seq = position_ids.shape
    table = _build_table(base, beta)
    idx3 = position_ids.reshape(_NW, _NCHUNK, _C).astype(jnp.int32)
    out = _make_gather()(table, idx3)
    return out.reshape(batch, seq, EMBEDDING_DIM)


# speedup vs baseline: 6.4254x; 1.6790x over previous
import functools
import math

import jax
import jax.numpy as jnp
import numpy as np
from jax import lax
from jax.experimental import pallas as pl
from jax.experimental.pallas import tpu as pltpu
from jax.experimental.pallas import tpu_sc as plsc

NUM_EMBEDDINGS = 8192
EMBEDDING_DIM = 1024
HALF_DIM = EMBEDDING_DIM // 2
_DT = math.log(10000.0) / (HALF_DIM - 1)
_DIV_FULL = np.exp(-_DT * (np.arange(EMBEDDING_DIM) // 2)).astype(np.float32)[None, :]

_ROWS_BLK = 1024
_GRID = NUM_EMBEDDINGS // _ROWS_BLK


_QR = 32


def _table_body(base_ref, beta_ref, div_ref, out_ref):
    i = pl.program_id(0)
    p0 = (i * _ROWS_BLK).astype(jnp.float32)
    freq = base_ref[0, 0] * div_ref[...] + beta_ref[0, 0]
    t = jax.lax.broadcasted_iota(jnp.int32, (_QR, EMBEDDING_DIM), 0).astype(
        jnp.float32) * freq
    a = p0 * freq + t * float(_QR)
    col = jax.lax.broadcasted_iota(jnp.int32, (_QR, EMBEDDING_DIM), 1)
    odd = col % 2 == 1
    ac, asn = jnp.cos(a), jnp.sin(a)
    bc = jnp.where(odd, jnp.cos(t), 0.0)
    bs = jnp.where(odd, jnp.sin(t), 0.0)

    def expand_q(x):
        return jnp.broadcast_to(x[:, None, :], (_QR, _QR, EMBEDDING_DIM)).reshape(
            _ROWS_BLK, EMBEDDING_DIM)

    def expand_r(x):
        return jnp.broadcast_to(x[None, :, :], (_QR, _QR, EMBEDDING_DIM)).reshape(
            _ROWS_BLK, EMBEDDING_DIM)

    out_ref[...] = expand_q(ac) * expand_r(bc) - expand_q(asn) * expand_r(bs)


def _build_table(base, beta):
    return pl.pallas_call(
        _table_body,
        grid=(_GRID,),
        in_specs=[
            pl.BlockSpec(memory_space=pltpu.SMEM),
            pl.BlockSpec(memory_space=pltpu.SMEM),
            pl.BlockSpec((1, EMBEDDING_DIM), lambda i: (0, 0)),
        ],
        out_specs=pl.BlockSpec((_ROWS_BLK, EMBEDDING_DIM), lambda i: (i, 0)),
        out_shape=jax.ShapeDtypeStruct((NUM_EMBEDDINGS, EMBEDDING_DIM), jnp.float32),
    )(base.reshape(1, 1), beta.reshape(1, 1), jnp.asarray(_DIV_FULL))


_NW = 32
_B = 4 * 8192
_BPW = _B // _NW
_C = 32
_NCHUNK = _BPW // _C


@functools.cache
def _make_gather():
    mesh = plsc.VectorSubcoreMesh(core_axis_name="c", subcore_axis_name="s")

    @functools.partial(
        pl.kernel,
        mesh=mesh,
        out_type=jax.ShapeDtypeStruct((_B, EMBEDDING_DIM), jnp.float32),
        scratch_types=[
            pltpu.VMEM((_NCHUNK, _C), jnp.int32),
            pltpu.VMEM((_C, EMBEDDING_DIM), jnp.float32),
            pltpu.VMEM((_C, EMBEDDING_DIM), jnp.float32),
            pltpu.SemaphoreType.DMA,
        ],
    )
    def gather(table_hbm, idx_hbm, out_hbm, idx_v, buf0, buf1, gsem):
        wid = lax.axis_index("s") * 2 + lax.axis_index("c")
        base_row = wid * _BPW
        pltpu.sync_copy(idx_hbm.at[wid], idx_v)
        bufs = (buf0, buf1)
        pltpu.async_copy(table_hbm.at[idx_v.at[0]], buf0, gsem).wait()
        for j in range(_NCHUNK):
            cur = bufs[j % 2]
            nxt = bufs[(j + 1) % 2]
            if j + 1 < _NCHUNK:
                g = pltpu.async_copy(table_hbm.at[idx_v.at[j + 1]], nxt, gsem)
            pltpu.sync_copy(cur, out_hbm.at[pl.ds(base_row + j * _C, _C)])
            if j + 1 < _NCHUNK:
                g.wait()

    return gather


def kernel(position_ids, base, alpha, beta):
    del alpha
    batch, seq = position_ids.shape
    table = _build_table(base, beta)
    idx3 = position_ids.reshape(_NW, _NCHUNK, _C).astype(jnp.int32)
    out = _make_gather()(table, idx3)
    return out.reshape(batch, seq, EMBEDDING_DIM)


# speedup vs baseline: 6.6939x; 1.0418x over previous
import functools
import math

import jax
import jax.numpy as jnp
import numpy as np
from jax import lax
from jax.experimental import pallas as pl
from jax.experimental.pallas import tpu as pltpu
from jax.experimental.pallas import tpu_sc as plsc

NUM_EMBEDDINGS = 8192
EMBEDDING_DIM = 1024
HALF_DIM = EMBEDDING_DIM // 2
_DT = math.log(10000.0) / (HALF_DIM - 1)
_DIV_FULL = np.exp(-_DT * (np.arange(EMBEDDING_DIM) // 2)).astype(np.float32)[None, :]

_ROWS_BLK = 1024
_GRID = NUM_EMBEDDINGS // _ROWS_BLK


_QR = 32


def _table_body(base_ref, beta_ref, div_ref, out_ref):
    i = pl.program_id(0)
    p0 = (i * _ROWS_BLK).astype(jnp.float32)
    freq = base_ref[0, 0] * div_ref[...] + beta_ref[0, 0]
    t = jax.lax.broadcasted_iota(jnp.int32, (_QR, EMBEDDING_DIM), 0).astype(
        jnp.float32) * freq
    a = p0 * freq + t * float(_QR)
    col = jax.lax.broadcasted_iota(jnp.int32, (_QR, EMBEDDING_DIM), 1)
    odd = col % 2 == 1
    ac, asn = jnp.cos(a), jnp.sin(a)
    bc = jnp.where(odd, jnp.cos(t), 0.0)
    bs = jnp.where(odd, jnp.sin(t), 0.0)

    def expand_q(x):
        return jnp.broadcast_to(x[:, None, :], (_QR, _QR, EMBEDDING_DIM)).reshape(
            _ROWS_BLK, EMBEDDING_DIM)

    def expand_r(x):
        return jnp.broadcast_to(x[None, :, :], (_QR, _QR, EMBEDDING_DIM)).reshape(
            _ROWS_BLK, EMBEDDING_DIM)

    out_ref[...] = expand_q(ac) * expand_r(bc) - expand_q(asn) * expand_r(bs)


def _build_table(base, beta):
    return pl.pallas_call(
        _table_body,
        grid=(_GRID,),
        in_specs=[
            pl.BlockSpec(memory_space=pltpu.SMEM),
            pl.BlockSpec(memory_space=pltpu.SMEM),
            pl.BlockSpec((1, EMBEDDING_DIM), lambda i: (0, 0)),
        ],
        out_specs=pl.BlockSpec((_ROWS_BLK, EMBEDDING_DIM), lambda i: (i, 0)),
        out_shape=jax.ShapeDtypeStruct((NUM_EMBEDDINGS, EMBEDDING_DIM), jnp.float32),
    )(base.reshape(1, 1), beta.reshape(1, 1), jnp.asarray(_DIV_FULL))


_NW = 32
_B = 4 * 8192
_BPW = _B // _NW
_C = 32
_NCHUNK = _BPW // _C


_DEPTH = 3


@functools.cache
def _make_gather():
    mesh = plsc.VectorSubcoreMesh(core_axis_name="c", subcore_axis_name="s")

    @functools.partial(
        pl.kernel,
        mesh=mesh,
        out_type=jax.ShapeDtypeStruct((_B, EMBEDDING_DIM), jnp.float32),
        scratch_types=[
            pltpu.VMEM((_NCHUNK, _C), jnp.int32),
        ]
        + [pltpu.VMEM((_C, EMBEDDING_DIM), jnp.float32)] * _DEPTH
        + [pltpu.SemaphoreType.DMA] * (2 * _DEPTH),
    )
    def gather(table_hbm, idx_hbm, out_hbm, idx_v, *rest):
        bufs = rest[:_DEPTH]
        gsems = rest[_DEPTH:2 * _DEPTH]
        ssems = rest[2 * _DEPTH:]
        wid = lax.axis_index("s") * 2 + lax.axis_index("c")
        base_row = wid * _BPW
        pltpu.sync_copy(idx_hbm.at[wid], idx_v)
        gathers = [None] * _NCHUNK
        for j in range(min(_DEPTH, _NCHUNK)):
            gathers[j] = pltpu.async_copy(
                table_hbm.at[idx_v.at[j]], bufs[j % _DEPTH], gsems[j % _DEPTH])
        for j in range(_NCHUNK):
            b = j % _DEPTH
            gathers[j].wait()
            s = pltpu.async_copy(
                bufs[b], out_hbm.at[pl.ds(base_row + j * _C, _C)], ssems[b])
            s.wait()
            if j + _DEPTH < _NCHUNK:
                gathers[j + _DEPTH] = pltpu.async_copy(
                    table_hbm.at[idx_v.at[j + _DEPTH]], bufs[b], gsems[b])

    return gather


def kernel(position_ids, base, alpha, beta):
    del alpha
    batch, seq = position_ids.shape
    table = _build_table(base, beta)
    idx3 = position_ids.reshape(_NW, _NCHUNK, _C).astype(jnp.int32)
    out = _make_gather()(table, idx3)
    return out.reshape(batch, seq, EMBEDDING_DIM)
